# baseline (speedup 1.0000x reference)
import functools

import jax
import jax.numpy as jnp
from jax import lax
from jax.experimental import pallas as pl
from jax.experimental.pallas import tpu as pltpu
from jax.experimental.pallas import tpu_sc as plsc

CHUNK = 128


@functools.lru_cache(maxsize=None)
def _make_gather(B, V, D):
    info = plsc.get_sparse_core_info()
    NC, NS = info.num_cores, info.num_subcores
    NW = NC * NS
    assert B % (NW * CHUNK) == 0
    b_per_w = B // NW
    n_chunks = b_per_w // CHUNK
    mesh = plsc.VectorSubcoreMesh(core_axis_name="c", subcore_axis_name="s")

    @functools.partial(
        pl.kernel,
        mesh=mesh,
        out_type=jax.ShapeDtypeStruct((B, D), jnp.float32),
        scratch_types=[
            pltpu.VMEM((CHUNK,), jnp.int32),
            pltpu.VMEM((CHUNK, D), jnp.float32),
            pltpu.SemaphoreType.DMA,
        ],
        compiler_params=pltpu.CompilerParams(use_tc_tiling_on_sc=False),
    )
    def gather_kernel(idx_hbm, table_hbm, out_hbm, idx_v, rows_v, sem):
        wid = lax.axis_index("s") * NC + lax.axis_index("c")
        base = wid * b_per_w

        def body(c, carry):
            off = base + c * CHUNK
            pltpu.sync_copy(idx_hbm.at[pl.ds(off, CHUNK)], idx_v)
            pltpu.async_copy(table_hbm.at[idx_v], rows_v, sem).wait()
            pltpu.sync_copy(rows_v, out_hbm.at[pl.ds(off, CHUNK), :])
            return carry

        lax.fori_loop(0, n_chunks, body, 0)

    return gather_kernel


def kernel(x, table):
    B, S = x.shape
    V, D = table.shape
    flat = x.reshape(B * S).astype(jnp.int32)
    out = _make_gather(B * S, V, D)(flat, table)
    return out.reshape(B, S, D)


# speedup vs baseline: 1.0822x; 1.0822x over previous
import functools

import jax
import jax.numpy as jnp
from jax import lax
from jax.experimental import pallas as pl
from jax.experimental.pallas import tpu as pltpu
from jax.experimental.pallas import tpu_sc as plsc

CHUNK = 128


@functools.lru_cache(maxsize=None)
def _make_gather(B, V, D):
    info = plsc.get_sparse_core_info()
    NC, NS = info.num_cores, info.num_subcores
    NW = NC * NS
    assert B % (NW * 2 * CHUNK) == 0
    b_per_w = B // NW
    n_pairs = b_per_w // (2 * CHUNK)
    mesh = plsc.VectorSubcoreMesh(core_axis_name="c", subcore_axis_name="s")

    @functools.partial(
        pl.kernel,
        mesh=mesh,
        out_type=jax.ShapeDtypeStruct((B, D), jnp.float32),
        scratch_types=[
            pltpu.VMEM((b_per_w,), jnp.int32),
            pltpu.VMEM((CHUNK, D), jnp.float32),
            pltpu.VMEM((CHUNK, D), jnp.float32),
            pltpu.SemaphoreType.DMA,
            pltpu.SemaphoreType.DMA,
            pltpu.SemaphoreType.DMA,
            pltpu.SemaphoreType.DMA,
        ],
        compiler_params=pltpu.CompilerParams(use_tc_tiling_on_sc=False),
    )
    def gather_kernel(idx_hbm, table_hbm, out_hbm, idx_v, rows0, rows1,
                      gs0, gs1, ss0, ss1):
        wid = lax.axis_index("s") * NC + lax.axis_index("c")
        base = wid * b_per_w
        pltpu.sync_copy(idx_hbm.at[pl.ds(base, b_per_w)], idx_v)

        def start_gather(c, buf, sem):
            pltpu.async_copy(
                table_hbm.at[idx_v.at[pl.ds(c * CHUNK, CHUNK)]], buf, sem)

        def wait_gather(c, buf, sem):
            pltpu.make_async_copy(
                table_hbm.at[idx_v.at[pl.ds(c * CHUNK, CHUNK)]], buf,
                sem).wait()

        def start_scatter(c, buf, sem):
            pltpu.async_copy(
                buf, out_hbm.at[pl.ds(base + c * CHUNK, CHUNK), :], sem)

        def wait_scatter(c, buf, sem):
            pltpu.make_async_copy(
                buf, out_hbm.at[pl.ds(base + c * CHUNK, CHUNK), :],
                sem).wait()

        start_gather(0, rows0, gs0)
        start_gather(1, rows1, gs1)
        wait_gather(0, rows0, gs0)
        start_scatter(0, rows0, ss0)
        wait_gather(1, rows1, gs1)
        start_scatter(1, rows1, ss1)

        def body(p, carry):
            c0 = 2 * p
            c1 = c0 + 1
            wait_scatter(c0 - 2, rows0, ss0)
            start_gather(c0, rows0, gs0)
            wait_scatter(c1 - 2, rows1, ss1)
            start_gather(c1, rows1, gs1)
            wait_gather(c0, rows0, gs0)
            start_scatter(c0, rows0, ss0)
            wait_gather(c1, rows1, gs1)
            start_scatter(c1, rows1, ss1)
            return carry

        lax.fori_loop(1, n_pairs, body, 0)
        last = 2 * (n_pairs - 1)
        wait_scatter(last, rows0, ss0)
        wait_scatter(last + 1, rows1, ss1)

    return gather_kernel


def kernel(x, table):
    B, S = x.shape
    V, D = table.shape
    flat = x.reshape(B * S).astype(jnp.int32)
    out = _make_gather(B * S, V, D)(flat, table)
    return out.reshape(B, S, D)


# speedup vs baseline: 1.8838x; 1.7407x over previous
import functools

import jax
import jax.numpy as jnp
from jax import lax
from jax.experimental import pallas as pl
from jax.experimental.pallas import tpu as pltpu
from jax.experimental.pallas import tpu_sc as plsc

CHUNK = 128
DP = 256


@functools.lru_cache(maxsize=None)
def _make_gather(B, V):
    info = plsc.get_sparse_core_info()
    NC, NS = info.num_cores, info.num_subcores
    NW = NC * NS
    assert B % (NW * 2 * CHUNK) == 0
    b_per_w = B // NW
    n_pairs = b_per_w // (2 * CHUNK)
    mesh = plsc.VectorSubcoreMesh(core_axis_name="c", subcore_axis_name="s")

    @functools.partial(
        pl.kernel,
        mesh=mesh,
        out_type=jax.ShapeDtypeStruct((B, DP), jnp.float32),
        scratch_types=[
            pltpu.VMEM((b_per_w,), jnp.int32),
            pltpu.VMEM((CHUNK, DP), jnp.float32),
            pltpu.VMEM((CHUNK, DP), jnp.float32),
            pltpu.SemaphoreType.DMA,
            pltpu.SemaphoreType.DMA,
            pltpu.SemaphoreType.DMA,
            pltpu.SemaphoreType.DMA,
        ],
    )
    def gather_kernel(idx_hbm, table_hbm, out_hbm, idx_v, rows0, rows1,
                      gs0, gs1, ss0, ss1):
        wid = lax.axis_index("s") * NC + lax.axis_index("c")
        base = wid * b_per_w
        pltpu.sync_copy(idx_hbm.at[pl.ds(base, b_per_w)], idx_v)

        def start_gather(c, buf, sem):
            pltpu.async_copy(
                table_hbm.at[idx_v.at[pl.ds(c * CHUNK, CHUNK)]], buf, sem)

        def wait_gather(c, buf, sem):
            pltpu.make_async_copy(
                table_hbm.at[idx_v.at[pl.ds(c * CHUNK, CHUNK)]], buf,
                sem).wait()

        def start_scatter(c, buf, sem):
            pltpu.async_copy(
                buf, out_hbm.at[pl.ds(base + c * CHUNK, CHUNK), :], sem)

        def wait_scatter(c, buf, sem):
            pltpu.make_async_copy(
                buf, out_hbm.at[pl.ds(base + c * CHUNK, CHUNK), :],
                sem).wait()

        start_gather(0, rows0, gs0)
        start_gather(1, rows1, gs1)
        wait_gather(0, rows0, gs0)
        start_scatter(0, rows0, ss0)
        wait_gather(1, rows1, gs1)
        start_scatter(1, rows1, ss1)

        def body(p, carry):
            c0 = 2 * p
            c1 = c0 + 1
            wait_scatter(c0 - 2, rows0, ss0)
            start_gather(c0, rows0, gs0)
            wait_scatter(c1 - 2, rows1, ss1)
            start_gather(c1, rows1, gs1)
            wait_gather(c0, rows0, gs0)
            start_scatter(c0, rows0, ss0)
            wait_gather(c1, rows1, gs1)
            start_scatter(c1, rows1, ss1)
            return carry

        lax.fori_loop(1, n_pairs, body, 0)
        last = 2 * (n_pairs - 1)
        wait_scatter(last, rows0, ss0)
        wait_scatter(last + 1, rows1, ss1)

    return gather_kernel


def kernel(x, table):
    B, S = x.shape
    V, D = table.shape
    flat = x.reshape(B * S).astype(jnp.int32)
    table_p = jnp.pad(table, ((0, 0), (0, DP - D)))
    out = _make_gather(B * S, V)(flat, table_p)
    return out[:, :D].reshape(B, S, D)
